# baseline (speedup 1.0000x reference)
import jax
import jax.numpy as jnp
from jax import lax
from jax.experimental import pallas as pl
from jax.experimental.pallas import tpu as pltpu
from jax.experimental.pallas import tpu_sc as plsc

N = 10000
D = 128
H = 128
HH = H // 2
NACC = 10240
C = 128
NCHUNK = 84
NTILES = 32
EP = NTILES * NCHUNK * C
RPT = NACC // 16


def _tc_pre_body(x_ref, w_ref, asv_ref, adv_ref,
                 hlo_ref, hhi_ref, as_ref, ad_ref, m_ref):
    h = jnp.dot(x_ref[...], w_ref[...], preferred_element_type=jnp.float32)
    hlo_ref[...] = h[:, :HH]
    hhi_ref[...] = h[:, HH:]
    a_s = jnp.dot(h, asv_ref[...], preferred_element_type=jnp.float32)
    a_d = jnp.dot(h, adv_ref[...], preferred_element_type=jnp.float32)
    as_ref[...] = a_s
    ad_ref[...] = a_d
    m = jnp.maximum(jnp.max(a_s) + jnp.max(a_d), 0.0)
    m_ref[...] = jnp.reshape(m, (1, 1))


_tc_pre = pl.pallas_call(
    _tc_pre_body,
    out_shape=[
        jax.ShapeDtypeStruct((N, HH), jnp.float32),
        jax.ShapeDtypeStruct((N, HH), jnp.float32),
        jax.ShapeDtypeStruct((N, 1), jnp.float32),
        jax.ShapeDtypeStruct((N, 1), jnp.float32),
        jax.ShapeDtypeStruct((1, 1), jnp.float32),
    ],
)


def _tc_mid_body(alo_ref, ahi_ref, den_ref, b_ref, a_ref, w_ref,
                 asv_ref, adv_ref, hlo_ref, hhi_ref, as_ref, ad_ref, m_ref):
    dens = den_ref[0, :N, :] + den_ref[1, :N, :] + 1e-16
    vlo = (alo_ref[0, :N, :] + alo_ref[1, :N, :]) / dens + b_ref[:, :HH]
    vhi = (ahi_ref[0, :N, :] + ahi_ref[1, :N, :]) / dens + b_ref[:, HH:]
    a = a_ref[0, 0]
    f = jnp.concatenate(
        [jnp.where(vlo >= 0, vlo, a * vlo), jnp.where(vhi >= 0, vhi, a * vhi)],
        axis=1)
    h2 = jnp.dot(f, w_ref[...], preferred_element_type=jnp.float32)
    hlo_ref[...] = h2[:, :HH]
    hhi_ref[...] = h2[:, HH:]
    a_s = jnp.dot(h2, asv_ref[...], preferred_element_type=jnp.float32)
    a_d = jnp.dot(h2, adv_ref[...], preferred_element_type=jnp.float32)
    as_ref[...] = a_s
    ad_ref[...] = a_d
    m = jnp.maximum(jnp.max(a_s) + jnp.max(a_d), 0.0)
    m_ref[...] = jnp.reshape(m, (1, 1))


_tc_mid = pl.pallas_call(
    _tc_mid_body,
    compiler_params=pltpu.CompilerParams(vmem_limit_bytes=100 * 1024 * 1024),
    out_shape=[
        jax.ShapeDtypeStruct((N, HH), jnp.float32),
        jax.ShapeDtypeStruct((N, HH), jnp.float32),
        jax.ShapeDtypeStruct((N, 1), jnp.float32),
        jax.ShapeDtypeStruct((N, 1), jnp.float32),
        jax.ShapeDtypeStruct((1, 1), jnp.float32),
    ],
)


def _tc_post_body(alo_ref, ahi_ref, den_ref, b_ref, a_ref, out_ref):
    dens = den_ref[0, :N, :] + den_ref[1, :N, :] + 1e-16
    vlo = (alo_ref[0, :N, :] + alo_ref[1, :N, :]) / dens + b_ref[:, :HH]
    vhi = (ahi_ref[0, :N, :] + ahi_ref[1, :N, :]) / dens + b_ref[:, HH:]
    a = a_ref[0, 0]
    out_ref[...] = jnp.concatenate(
        [jnp.where(vlo >= 0, vlo, a * vlo), jnp.where(vhi >= 0, vhi, a * vhi)],
        axis=1)


_tc_post = pl.pallas_call(
    _tc_post_body,
    compiler_params=pltpu.CompilerParams(vmem_limit_bytes=100 * 1024 * 1024),
    out_shape=jax.ShapeDtypeStruct((N, H), jnp.float32),
)


def _sc_edge_body(src_hbm, dst_hbm, hlo_hbm, hhi_hbm, as_hbm, ad_hbm, m_hbm,
                  alo_out, ahi_out, den_out,
                  asbuf, adbuf, srcbuf, dstbuf, exall, rows_a, rows_b,
                  zbuf, mbuf, acc_sh, den_sh, sem_g0, sem_g1):
    c = lax.axis_index("c")
    s = lax.axis_index("s")
    blk = c * 16 + s
    base = s * RPT

    zero16 = jnp.zeros((16,), jnp.float32)

    def _zero_rows_a():
        def _zrow(i, carry):
            for k in range(HH // 16):
                rows_a[i, pl.ds(k * 16, 16)] = zero16
            return carry
        lax.fori_loop(0, C, _zrow, 0)

    def _zero_acc_slice():
        for r in range(4):
            pltpu.sync_copy(rows_a, acc_sh.at[pl.ds(base + r * C, C)])
        pltpu.sync_copy(rows_a.at[pl.ds(0, RPT - 4 * C)],
                        acc_sh.at[pl.ds(base + 4 * C, RPT - 4 * C)])

    _zero_rows_a()
    _zero_acc_slice()
    for k in range(RPT // 16):
        zbuf[pl.ds(k * 16, 16)] = zero16
    pltpu.sync_copy(zbuf, den_sh.at[pl.ds(base, RPT)])

    pltpu.sync_copy(as_hbm, asbuf)
    pltpu.sync_copy(ad_hbm, adbuf)
    pltpu.sync_copy(src_hbm.at[blk], srcbuf)
    pltpu.sync_copy(dst_hbm.at[blk], dstbuf)
    pltpu.sync_copy(m_hbm, mbuf)
    plsc.subcore_barrier()
    mv = mbuf[...]

    rows = (rows_a, rows_b)
    sg = (sem_g0, sem_g1)

    def _run_pass(h_hbm, compute_ex):
        def _do_chunk(j, p):
            if compute_ex:
                for g in range(C // 16):
                    sidx = srcbuf[j, pl.ds(g * 16, 16)]
                    didx = dstbuf[j, pl.ds(g * 16, 16)]
                    z = (plsc.load_gather(asbuf, [sidx])
                         + plsc.load_gather(adbuf, [didx]))
                    e = jnp.where(z >= 0, z, jnp.float32(0.2) * z)
                    exall[j, pl.ds(g * 16, 16)] = jnp.exp(e - mv)
                pltpu.sync_copy(exall.at[j], den_sh.at[dstbuf.at[j]], add=True)

            def _scale(g, carry):
                ex16 = exall[j, pl.ds(g * 16, 16)]
                for l in range(16):
                    ex = ex16[l]
                    i = g * 16 + l
                    for k in range(HH // 16):
                        rows[p][i, pl.ds(k * 16, 16)] = (
                            rows[p][i, pl.ds(k * 16, 16)] * ex)
                return carry

            lax.fori_loop(0, C // 16, _scale, 0)
            pltpu.sync_copy(rows[p], acc_sh.at[dstbuf.at[j]], add=True)

        pltpu.async_copy(h_hbm.at[srcbuf.at[0]], rows[0], sg[0])

        def _outer(jj, carry):
            j = jj * 2
            pltpu.async_copy(h_hbm.at[srcbuf.at[j + 1]], rows[1], sg[1])
            pltpu.make_async_copy(
                h_hbm.at[srcbuf.at[j]], rows[0], sg[0]).wait()
            _do_chunk(j, 0)

            @pl.when(jj < NCHUNK // 2 - 1)
            def _():
                pltpu.async_copy(h_hbm.at[srcbuf.at[j + 2]], rows[0], sg[0])

            pltpu.make_async_copy(
                h_hbm.at[srcbuf.at[j + 1]], rows[1], sg[1]).wait()
            _do_chunk(j + 1, 1)
            return carry

        lax.fori_loop(0, NCHUNK // 2, _outer, 0)

    _run_pass(hlo_hbm, compute_ex=True)
    plsc.subcore_barrier()
    pltpu.sync_copy(acc_sh.at[pl.ds(base, RPT)],
                    alo_out.at[c].at[pl.ds(base, RPT)])
    pltpu.sync_copy(den_sh.at[pl.ds(base, RPT)],
                    den_out.at[c].at[pl.ds(base, RPT)])
    _zero_rows_a()
    _zero_acc_slice()
    plsc.subcore_barrier()
    _run_pass(hhi_hbm, compute_ex=False)
    plsc.subcore_barrier()
    pltpu.sync_copy(acc_sh.at[pl.ds(base, RPT)],
                    ahi_out.at[c].at[pl.ds(base, RPT)])


_sc_edge = pl.kernel(
    _sc_edge_body,
    out_type=[
        jax.ShapeDtypeStruct((2, NACC, HH), jnp.float32),
        jax.ShapeDtypeStruct((2, NACC, HH), jnp.float32),
        jax.ShapeDtypeStruct((2, NACC), jnp.float32),
    ],
    mesh=plsc.VectorSubcoreMesh(core_axis_name="c", subcore_axis_name="s",
                                num_cores=2, num_subcores=16),
    compiler_params=pltpu.CompilerParams(needs_layout_passes=False,
                                         use_tc_tiling_on_sc=False),
    scratch_types=[
        pltpu.VMEM((NACC,), jnp.float32),
        pltpu.VMEM((NACC,), jnp.float32),
        pltpu.VMEM((NCHUNK, C), jnp.int32),
        pltpu.VMEM((NCHUNK, C), jnp.int32),
        pltpu.VMEM((NCHUNK, C), jnp.float32),
        pltpu.VMEM((C, HH), jnp.float32),
        pltpu.VMEM((C, HH), jnp.float32),
        pltpu.VMEM((RPT,), jnp.float32),
        pltpu.VMEM((16,), jnp.float32),
        pltpu.VMEM_SHARED((NACC, HH), jnp.float32),
        pltpu.VMEM_SHARED((NACC,), jnp.float32),
        pltpu.SemaphoreType.DMA,
        pltpu.SemaphoreType.DMA,
    ],
)


def _pad_nodes(a):
    return jnp.zeros((NACC,), jnp.float32).at[:N].set(a[:, 0])


def kernel(x, edge_index, W1, att_src1, att_dst1, b1, W2, att_src2, att_dst2,
           b2, prelu_a):
    loop = jnp.arange(N, dtype=jnp.int32)
    npad = EP - (edge_index.shape[1] + N)
    src = jnp.concatenate(
        [edge_index[0], loop, jnp.zeros((npad,), jnp.int32)]
    ).reshape(NTILES, NCHUNK, C)
    pad_dst = N + jnp.arange(npad, dtype=jnp.int32) % (NACC - N)
    dst = jnp.concatenate(
        [edge_index[1], loop, pad_dst]
    ).reshape(NTILES, NCHUNK, C)

    a2 = prelu_a.reshape(1, 1)

    h1lo, h1hi, as1, ad1, m1 = _tc_pre(x, W1, att_src1.reshape(H, 1),
                                       att_dst1.reshape(H, 1))
    alo1, ahi1, den1 = _sc_edge(src, dst, h1lo, h1hi, _pad_nodes(as1),
                                _pad_nodes(ad1),
                                jnp.full((16,), m1[0, 0], jnp.float32))
    h2lo, h2hi, as2, ad2, m2 = _tc_mid(alo1, ahi1, den1.reshape(2, NACC, 1),
                                       b1.reshape(1, H), a2, W2,
                                       att_src2.reshape(H, 1),
                                       att_dst2.reshape(H, 1))
    alo2, ahi2, den2 = _sc_edge(src, dst, h2lo, h2hi, _pad_nodes(as2),
                                _pad_nodes(ad2),
                                jnp.full((16,), m2[0, 0], jnp.float32))
    return _tc_post(alo2, ahi2, den2.reshape(2, NACC, 1), b2.reshape(1, H), a2)


# speedup vs baseline: 2.0268x; 2.0268x over previous
import jax
import jax.numpy as jnp
from jax import lax
from jax.experimental import pallas as pl
from jax.experimental.pallas import tpu as pltpu
from jax.experimental.pallas import tpu_sc as plsc

N = 10000
D = 128
H = 128
HH = H // 2
NACC = 10240
C = 128
NCHUNK = 84
NTILES = 32
EP = NTILES * NCHUNK * C
RPT = NACC // 16


def _tc_pre_body(x_ref, w_ref, asv_ref, adv_ref,
                 hlo_ref, hhi_ref, as_ref, ad_ref, m_ref):
    h = jnp.dot(x_ref[...], w_ref[...], preferred_element_type=jnp.float32)
    hlo_ref[...] = h[:, :HH]
    hhi_ref[...] = h[:, HH:]
    a_s = jnp.dot(h, asv_ref[...], preferred_element_type=jnp.float32)
    a_d = jnp.dot(h, adv_ref[...], preferred_element_type=jnp.float32)
    as_ref[...] = a_s
    ad_ref[...] = a_d
    m = jnp.maximum(jnp.max(a_s) + jnp.max(a_d), 0.0)
    m_ref[...] = jnp.reshape(m, (1, 1))


_tc_pre = pl.pallas_call(
    _tc_pre_body,
    out_shape=[
        jax.ShapeDtypeStruct((N, HH), jnp.float32),
        jax.ShapeDtypeStruct((N, HH), jnp.float32),
        jax.ShapeDtypeStruct((N, 1), jnp.float32),
        jax.ShapeDtypeStruct((N, 1), jnp.float32),
        jax.ShapeDtypeStruct((1, 1), jnp.float32),
    ],
)


def _tc_mid_body(alo_ref, ahi_ref, den_ref, b_ref, a_ref, w_ref,
                 asv_ref, adv_ref, hlo_ref, hhi_ref, as_ref, ad_ref, m_ref):
    dens = den_ref[0, :N, :] + den_ref[1, :N, :] + 1e-16
    vlo = (alo_ref[0, :N, :] + alo_ref[1, :N, :]) / dens + b_ref[:, :HH]
    vhi = (ahi_ref[0, :N, :] + ahi_ref[1, :N, :]) / dens + b_ref[:, HH:]
    a = a_ref[0, 0]
    f = jnp.concatenate(
        [jnp.where(vlo >= 0, vlo, a * vlo), jnp.where(vhi >= 0, vhi, a * vhi)],
        axis=1)
    h2 = jnp.dot(f, w_ref[...], preferred_element_type=jnp.float32)
    hlo_ref[...] = h2[:, :HH]
    hhi_ref[...] = h2[:, HH:]
    a_s = jnp.dot(h2, asv_ref[...], preferred_element_type=jnp.float32)
    a_d = jnp.dot(h2, adv_ref[...], preferred_element_type=jnp.float32)
    as_ref[...] = a_s
    ad_ref[...] = a_d
    m = jnp.maximum(jnp.max(a_s) + jnp.max(a_d), 0.0)
    m_ref[...] = jnp.reshape(m, (1, 1))


_tc_mid = pl.pallas_call(
    _tc_mid_body,
    compiler_params=pltpu.CompilerParams(vmem_limit_bytes=100 * 1024 * 1024),
    out_shape=[
        jax.ShapeDtypeStruct((N, HH), jnp.float32),
        jax.ShapeDtypeStruct((N, HH), jnp.float32),
        jax.ShapeDtypeStruct((N, 1), jnp.float32),
        jax.ShapeDtypeStruct((N, 1), jnp.float32),
        jax.ShapeDtypeStruct((1, 1), jnp.float32),
    ],
)


def _tc_post_body(alo_ref, ahi_ref, den_ref, b_ref, a_ref, out_ref):
    dens = den_ref[0, :N, :] + den_ref[1, :N, :] + 1e-16
    vlo = (alo_ref[0, :N, :] + alo_ref[1, :N, :]) / dens + b_ref[:, :HH]
    vhi = (ahi_ref[0, :N, :] + ahi_ref[1, :N, :]) / dens + b_ref[:, HH:]
    a = a_ref[0, 0]
    out_ref[...] = jnp.concatenate(
        [jnp.where(vlo >= 0, vlo, a * vlo), jnp.where(vhi >= 0, vhi, a * vhi)],
        axis=1)


_tc_post = pl.pallas_call(
    _tc_post_body,
    compiler_params=pltpu.CompilerParams(vmem_limit_bytes=100 * 1024 * 1024),
    out_shape=jax.ShapeDtypeStruct((N, H), jnp.float32),
)


def _sc_edge_body(src_hbm, dst_hbm, hlo_hbm, hhi_hbm, as_hbm, ad_hbm, m_hbm,
                  alo_out, ahi_out, den_out,
                  asbuf, adbuf, srcbuf, dstbuf, exall, rows_a, rows_b,
                  zbuf, mbuf, acc_sh, den_sh, sem_g0, sem_g1):
    c = lax.axis_index("c")
    s = lax.axis_index("s")
    blk = c * 16 + s
    base = s * RPT

    zero16 = jnp.zeros((16,), jnp.float32)

    def _zero_rows_a():
        def _zrow(i, carry):
            for k in range(HH // 16):
                rows_a[i, pl.ds(k * 16, 16)] = zero16
            return carry
        lax.fori_loop(0, C, _zrow, 0)

    def _zero_acc_slice():
        for r in range(4):
            pltpu.sync_copy(rows_a, acc_sh.at[pl.ds(base + r * C, C)])
        pltpu.sync_copy(rows_a.at[pl.ds(0, RPT - 4 * C)],
                        acc_sh.at[pl.ds(base + 4 * C, RPT - 4 * C)])

    _zero_rows_a()
    _zero_acc_slice()
    for k in range(RPT // 16):
        zbuf[pl.ds(k * 16, 16)] = zero16
    pltpu.sync_copy(zbuf, den_sh.at[pl.ds(base, RPT)])

    pltpu.sync_copy(as_hbm, asbuf)
    pltpu.sync_copy(ad_hbm, adbuf)
    pltpu.sync_copy(src_hbm.at[blk], srcbuf)
    pltpu.sync_copy(dst_hbm.at[blk], dstbuf)
    pltpu.sync_copy(m_hbm, mbuf)
    plsc.subcore_barrier()
    mv = mbuf[...]

    rows = (rows_a, rows_b)
    sg = (sem_g0, sem_g1)

    def _run_pass(h_hbm, compute_ex):
        def _do_chunk(j, p):
            if compute_ex:
                for g in range(C // 16):
                    sidx = srcbuf[j, pl.ds(g * 16, 16)]
                    didx = dstbuf[j, pl.ds(g * 16, 16)]
                    z = (plsc.load_gather(asbuf, [sidx])
                         + plsc.load_gather(adbuf, [didx]))
                    e = jnp.where(z >= 0, z, jnp.float32(0.2) * z)
                    exall[j, pl.ds(g * 16, 16)] = jnp.exp(e - mv)
                pltpu.sync_copy(exall.at[j], den_sh.at[dstbuf.at[j]], add=True)

            def _scale(g, carry):
                ex16 = exall[j, pl.ds(g * 16, 16)]
                for l in range(16):
                    ex = ex16[l]
                    i = g * 16 + l
                    for k in range(HH // 16):
                        rows[p][i, pl.ds(k * 16, 16)] = (
                            rows[p][i, pl.ds(k * 16, 16)] * ex)
                return carry

            lax.fori_loop(0, C // 16, _scale, 0)
            pltpu.sync_copy(rows[p], acc_sh.at[dstbuf.at[j]], add=True)

        pltpu.async_copy(h_hbm.at[srcbuf.at[0]], rows[0], sg[0])

        def _outer(jj, carry):
            j = jj * 2
            pltpu.async_copy(h_hbm.at[srcbuf.at[j + 1]], rows[1], sg[1])
            pltpu.make_async_copy(
                h_hbm.at[srcbuf.at[j]], rows[0], sg[0]).wait()
            _do_chunk(j, 0)

            @pl.when(jj < NCHUNK // 2 - 1)
            def _():
                pltpu.async_copy(h_hbm.at[srcbuf.at[j + 2]], rows[0], sg[0])

            pltpu.make_async_copy(
                h_hbm.at[srcbuf.at[j + 1]], rows[1], sg[1]).wait()
            _do_chunk(j + 1, 1)
            return carry

        lax.fori_loop(0, NCHUNK // 2, _outer, 0)

    _run_pass(hlo_hbm, compute_ex=True)
    plsc.subcore_barrier()
    pltpu.sync_copy(acc_sh.at[pl.ds(base, RPT)],
                    alo_out.at[c].at[pl.ds(base, RPT)])
    pltpu.sync_copy(den_sh.at[pl.ds(base, RPT)],
                    den_out.at[c].at[pl.ds(base, RPT)])
    _zero_rows_a()
    _zero_acc_slice()
    plsc.subcore_barrier()
    _run_pass(hhi_hbm, compute_ex=False)
    plsc.subcore_barrier()
    pltpu.sync_copy(acc_sh.at[pl.ds(base, RPT)],
                    ahi_out.at[c].at[pl.ds(base, RPT)])


_sc_edge = pl.kernel(
    _sc_edge_body,
    out_type=[
        jax.ShapeDtypeStruct((2, NACC, HH), jnp.float32),
        jax.ShapeDtypeStruct((2, NACC, HH), jnp.float32),
        jax.ShapeDtypeStruct((2, NACC), jnp.float32),
    ],
    mesh=plsc.VectorSubcoreMesh(core_axis_name="c", subcore_axis_name="s",
                                num_cores=2, num_subcores=16),
    compiler_params=pltpu.CompilerParams(needs_layout_passes=False,
                                         use_tc_tiling_on_sc=False),
    scratch_types=[
        pltpu.VMEM((NACC,), jnp.float32),
        pltpu.VMEM((NACC,), jnp.float32),
        pltpu.VMEM((NCHUNK, C), jnp.int32),
        pltpu.VMEM((NCHUNK, C), jnp.int32),
        pltpu.VMEM((NCHUNK, C), jnp.float32),
        pltpu.VMEM((C, HH), jnp.float32),
        pltpu.VMEM((C, HH), jnp.float32),
        pltpu.VMEM((RPT,), jnp.float32),
        pltpu.VMEM((16,), jnp.float32),
        pltpu.VMEM_SHARED((NACC, HH), jnp.float32),
        pltpu.VMEM_SHARED((NACC,), jnp.float32),
        pltpu.SemaphoreType.DMA,
        pltpu.SemaphoreType.DMA,
    ],
)


def _pad_nodes(a):
    return jnp.zeros((NACC,), jnp.float32).at[:N].set(a[:, 0])


def kernel(x, edge_index, W1, att_src1, att_dst1, b1, W2, att_src2, att_dst2,
           b2, prelu_a):
    loop = jnp.arange(N, dtype=jnp.int32)
    npad = EP - (edge_index.shape[1] + N)
    pad_src = jnp.arange(npad, dtype=jnp.int32) % N
    src = jnp.concatenate(
        [edge_index[0], loop, pad_src]
    ).reshape(NTILES, NCHUNK, C)
    pad_dst = N + jnp.arange(npad, dtype=jnp.int32) % (NACC - N)
    dst = jnp.concatenate(
        [edge_index[1], loop, pad_dst]
    ).reshape(NTILES, NCHUNK, C)

    a2 = prelu_a.reshape(1, 1)

    h1lo, h1hi, as1, ad1, m1 = _tc_pre(x, W1, att_src1.reshape(H, 1),
                                       att_dst1.reshape(H, 1))
    alo1, ahi1, den1 = _sc_edge(src, dst, h1lo, h1hi, _pad_nodes(as1),
                                _pad_nodes(ad1),
                                jnp.full((16,), m1[0, 0], jnp.float32))
    h2lo, h2hi, as2, ad2, m2 = _tc_mid(alo1, ahi1, den1.reshape(2, NACC, 1),
                                       b1.reshape(1, H), a2, W2,
                                       att_src2.reshape(H, 1),
                                       att_dst2.reshape(H, 1))
    alo2, ahi2, den2 = _sc_edge(src, dst, h2lo, h2hi, _pad_nodes(as2),
                                _pad_nodes(ad2),
                                jnp.full((16,), m2[0, 0], jnp.float32))
    return _tc_post(alo2, ahi2, den2.reshape(2, NACC, 1), b2.reshape(1, H), a2)


# speedup vs baseline: 2.0634x; 1.0181x over previous
import jax
import jax.numpy as jnp
from jax import lax
from jax.experimental import pallas as pl
from jax.experimental.pallas import tpu as pltpu
from jax.experimental.pallas import tpu_sc as plsc

N = 10000
D = 128
H = 128
HH = H // 2
NACC = 10240
C = 128
NCHUNK = 84
NTILES = 32
EP = NTILES * NCHUNK * C
RPT = NACC // 16


def _tc_pre_body(x_ref, w_ref, asv_ref, adv_ref,
                 hlo_ref, hhi_ref, as_ref, ad_ref, m_ref):
    h = jnp.dot(x_ref[...], w_ref[...], preferred_element_type=jnp.float32)
    hlo_ref[...] = h[:, :HH]
    hhi_ref[...] = h[:, HH:]
    a_s = jnp.dot(h, asv_ref[...], preferred_element_type=jnp.float32)
    a_d = jnp.dot(h, adv_ref[...], preferred_element_type=jnp.float32)
    as_ref[...] = a_s
    ad_ref[...] = a_d
    m = jnp.maximum(jnp.max(a_s) + jnp.max(a_d), 0.0)
    m_ref[...] = jnp.reshape(m, (1, 1))


_tc_pre = pl.pallas_call(
    _tc_pre_body,
    out_shape=[
        jax.ShapeDtypeStruct((N, HH), jnp.float32),
        jax.ShapeDtypeStruct((N, HH), jnp.float32),
        jax.ShapeDtypeStruct((N, 1), jnp.float32),
        jax.ShapeDtypeStruct((N, 1), jnp.float32),
        jax.ShapeDtypeStruct((1, 1), jnp.float32),
    ],
)


def _tc_mid_body(alo_ref, ahi_ref, den_ref, b_ref, a_ref, w_ref,
                 asv_ref, adv_ref, hlo_ref, hhi_ref, as_ref, ad_ref, m_ref):
    dens = den_ref[0, :N, :] + den_ref[1, :N, :] + 1e-16
    vlo = (alo_ref[0, :N, :] + alo_ref[1, :N, :]) / dens + b_ref[:, :HH]
    vhi = (ahi_ref[0, :N, :] + ahi_ref[1, :N, :]) / dens + b_ref[:, HH:]
    a = a_ref[0, 0]
    f = jnp.concatenate(
        [jnp.where(vlo >= 0, vlo, a * vlo), jnp.where(vhi >= 0, vhi, a * vhi)],
        axis=1)
    h2 = jnp.dot(f, w_ref[...], preferred_element_type=jnp.float32)
    hlo_ref[...] = h2[:, :HH]
    hhi_ref[...] = h2[:, HH:]
    a_s = jnp.dot(h2, asv_ref[...], preferred_element_type=jnp.float32)
    a_d = jnp.dot(h2, adv_ref[...], preferred_element_type=jnp.float32)
    as_ref[...] = a_s
    ad_ref[...] = a_d
    m = jnp.maximum(jnp.max(a_s) + jnp.max(a_d), 0.0)
    m_ref[...] = jnp.reshape(m, (1, 1))


_tc_mid = pl.pallas_call(
    _tc_mid_body,
    compiler_params=pltpu.CompilerParams(vmem_limit_bytes=100 * 1024 * 1024),
    out_shape=[
        jax.ShapeDtypeStruct((N, HH), jnp.float32),
        jax.ShapeDtypeStruct((N, HH), jnp.float32),
        jax.ShapeDtypeStruct((N, 1), jnp.float32),
        jax.ShapeDtypeStruct((N, 1), jnp.float32),
        jax.ShapeDtypeStruct((1, 1), jnp.float32),
    ],
)


def _tc_post_body(alo_ref, ahi_ref, den_ref, b_ref, a_ref, out_ref):
    dens = den_ref[0, :N, :] + den_ref[1, :N, :] + 1e-16
    vlo = (alo_ref[0, :N, :] + alo_ref[1, :N, :]) / dens + b_ref[:, :HH]
    vhi = (ahi_ref[0, :N, :] + ahi_ref[1, :N, :]) / dens + b_ref[:, HH:]
    a = a_ref[0, 0]
    out_ref[...] = jnp.concatenate(
        [jnp.where(vlo >= 0, vlo, a * vlo), jnp.where(vhi >= 0, vhi, a * vhi)],
        axis=1)


_tc_post = pl.pallas_call(
    _tc_post_body,
    compiler_params=pltpu.CompilerParams(vmem_limit_bytes=100 * 1024 * 1024),
    out_shape=jax.ShapeDtypeStruct((N, H), jnp.float32),
)


def _sc_edge_body(src_hbm, dst_hbm, hlo_hbm, hhi_hbm, as_hbm, ad_hbm, m_hbm,
                  alo_out, ahi_out, den_out,
                  asbuf, adbuf, srcbuf, dstbuf, exall, rows_a, rows_b, rows_c,
                  zbuf, mbuf, acc_sh, den_sh,
                  sem_g0, sem_g1, sem_g2, sem_s0, sem_s1, sem_s2, sem_d):
    c = lax.axis_index("c")
    s = lax.axis_index("s")
    blk = c * 16 + s
    base = s * RPT

    zero16 = jnp.zeros((16,), jnp.float32)

    def _zero_rows_a():
        def _zrow(i, carry):
            for k in range(HH // 16):
                rows_a[i, pl.ds(k * 16, 16)] = zero16
            return carry
        lax.fori_loop(0, C, _zrow, 0)

    def _zero_acc_slice():
        for r in range(4):
            pltpu.sync_copy(rows_a, acc_sh.at[pl.ds(base + r * C, C)])
        pltpu.sync_copy(rows_a.at[pl.ds(0, RPT - 4 * C)],
                        acc_sh.at[pl.ds(base + 4 * C, RPT - 4 * C)])

    _zero_rows_a()
    _zero_acc_slice()
    for k in range(RPT // 16):
        zbuf[pl.ds(k * 16, 16)] = zero16
    pltpu.sync_copy(zbuf, den_sh.at[pl.ds(base, RPT)])

    pltpu.sync_copy(as_hbm, asbuf)
    pltpu.sync_copy(ad_hbm, adbuf)
    pltpu.sync_copy(src_hbm.at[blk], srcbuf)
    pltpu.sync_copy(dst_hbm.at[blk], dstbuf)
    pltpu.sync_copy(m_hbm, mbuf)
    plsc.subcore_barrier()
    mv = mbuf[...]

    rows = (rows_a, rows_b, rows_c)
    sg = (sem_g0, sem_g1, sem_g2)
    ss = (sem_s0, sem_s1, sem_s2)

    def _run_pass(h_hbm, compute_ex):
        def _phase(j, p):
            b2 = (p + 2) % 3

            @pl.when(j >= 1)
            def _():
                pltpu.make_async_copy(
                    rows[b2], acc_sh.at[dstbuf.at[j]], ss[b2]).wait()

            @pl.when(j + 2 < NCHUNK)
            def _():
                pltpu.async_copy(h_hbm.at[srcbuf.at[j + 2]], rows[b2], sg[b2])

            pltpu.make_async_copy(
                h_hbm.at[srcbuf.at[j]], rows[p], sg[p]).wait()

            if compute_ex:
                for g in range(C // 16):
                    sidx = srcbuf[j, pl.ds(g * 16, 16)]
                    didx = dstbuf[j, pl.ds(g * 16, 16)]
                    z = (plsc.load_gather(asbuf, [sidx])
                         + plsc.load_gather(adbuf, [didx]))
                    e = jnp.where(z >= 0, z, jnp.float32(0.2) * z)
                    exall[j, pl.ds(g * 16, 16)] = jnp.exp(e - mv)

                @pl.when(j >= 1)
                def _():
                    pltpu.make_async_copy(
                        exall.at[j], den_sh.at[dstbuf.at[j]], sem_d).wait()

                pltpu.async_copy(exall.at[j], den_sh.at[dstbuf.at[j]], sem_d,
                                 add=True)

            def _scale(g, carry):
                ex16 = exall[j, pl.ds(g * 16, 16)]
                for l in range(16):
                    ex = ex16[l]
                    i = g * 16 + l
                    for k in range(HH // 16):
                        rows[p][i, pl.ds(k * 16, 16)] = (
                            rows[p][i, pl.ds(k * 16, 16)] * ex)
                return carry

            lax.fori_loop(0, C // 16, _scale, 0)
            pltpu.async_copy(rows[p], acc_sh.at[dstbuf.at[j]], ss[p], add=True)

        pltpu.async_copy(h_hbm.at[srcbuf.at[0]], rows[0], sg[0])
        pltpu.async_copy(h_hbm.at[srcbuf.at[1]], rows[1], sg[1])

        def _outer(jj, carry):
            for p in range(3):
                _phase(jj * 3 + p, p)
            return carry

        lax.fori_loop(0, NCHUNK // 3, _outer, 0)
        pltpu.make_async_copy(
            rows[(NCHUNK - 1) % 3], acc_sh.at[dstbuf.at[0]],
            ss[(NCHUNK - 1) % 3]).wait()
        if compute_ex:
            pltpu.make_async_copy(
                exall.at[0], den_sh.at[dstbuf.at[0]], sem_d).wait()

    _run_pass(hlo_hbm, compute_ex=True)
    plsc.subcore_barrier()
    pltpu.sync_copy(acc_sh.at[pl.ds(base, RPT)],
                    alo_out.at[c].at[pl.ds(base, RPT)])
    pltpu.sync_copy(den_sh.at[pl.ds(base, RPT)],
                    den_out.at[c].at[pl.ds(base, RPT)])
    _zero_rows_a()
    _zero_acc_slice()
    plsc.subcore_barrier()
    _run_pass(hhi_hbm, compute_ex=False)
    plsc.subcore_barrier()
    pltpu.sync_copy(acc_sh.at[pl.ds(base, RPT)],
                    ahi_out.at[c].at[pl.ds(base, RPT)])


_sc_edge = pl.kernel(
    _sc_edge_body,
    out_type=[
        jax.ShapeDtypeStruct((2, NACC, HH), jnp.float32),
        jax.ShapeDtypeStruct((2, NACC, HH), jnp.float32),
        jax.ShapeDtypeStruct((2, NACC), jnp.float32),
    ],
    mesh=plsc.VectorSubcoreMesh(core_axis_name="c", subcore_axis_name="s",
                                num_cores=2, num_subcores=16),
    compiler_params=pltpu.CompilerParams(needs_layout_passes=False,
                                         use_tc_tiling_on_sc=False),
    scratch_types=[
        pltpu.VMEM((NACC,), jnp.float32),
        pltpu.VMEM((NACC,), jnp.float32),
        pltpu.VMEM((NCHUNK, C), jnp.int32),
        pltpu.VMEM((NCHUNK, C), jnp.int32),
        pltpu.VMEM((NCHUNK, C), jnp.float32),
        pltpu.VMEM((C, HH), jnp.float32),
        pltpu.VMEM((C, HH), jnp.float32),
        pltpu.VMEM((C, HH), jnp.float32),
        pltpu.VMEM((RPT,), jnp.float32),
        pltpu.VMEM((16,), jnp.float32),
        pltpu.VMEM_SHARED((NACC, HH), jnp.float32),
        pltpu.VMEM_SHARED((NACC,), jnp.float32),
        pltpu.SemaphoreType.DMA,
        pltpu.SemaphoreType.DMA,
        pltpu.SemaphoreType.DMA,
        pltpu.SemaphoreType.DMA,
        pltpu.SemaphoreType.DMA,
        pltpu.SemaphoreType.DMA,
        pltpu.SemaphoreType.DMA,
    ],
)


def _pad_nodes(a):
    return jnp.zeros((NACC,), jnp.float32).at[:N].set(a[:, 0])


def kernel(x, edge_index, W1, att_src1, att_dst1, b1, W2, att_src2, att_dst2,
           b2, prelu_a):
    loop = jnp.arange(N, dtype=jnp.int32)
    npad = EP - (edge_index.shape[1] + N)
    pad_src = jnp.arange(npad, dtype=jnp.int32) % N
    src = jnp.concatenate(
        [edge_index[0], loop, pad_src]
    ).reshape(NTILES, NCHUNK, C)
    pad_dst = N + jnp.arange(npad, dtype=jnp.int32) % (NACC - N)
    dst = jnp.concatenate(
        [edge_index[1], loop, pad_dst]
    ).reshape(NTILES, NCHUNK, C)

    a2 = prelu_a.reshape(1, 1)

    h1lo, h1hi, as1, ad1, m1 = _tc_pre(x, W1, att_src1.reshape(H, 1),
                                       att_dst1.reshape(H, 1))
    alo1, ahi1, den1 = _sc_edge(src, dst, h1lo, h1hi, _pad_nodes(as1),
                                _pad_nodes(ad1),
                                jnp.full((16,), m1[0, 0], jnp.float32))
    h2lo, h2hi, as2, ad2, m2 = _tc_mid(alo1, ahi1, den1.reshape(2, NACC, 1),
                                       b1.reshape(1, H), a2, W2,
                                       att_src2.reshape(H, 1),
                                       att_dst2.reshape(H, 1))
    alo2, ahi2, den2 = _sc_edge(src, dst, h2lo, h2hi, _pad_nodes(as2),
                                _pad_nodes(ad2),
                                jnp.full((16,), m2[0, 0], jnp.float32))
    return _tc_post(alo2, ahi2, den2.reshape(2, NACC, 1), b2.reshape(1, H), a2)


# speedup vs baseline: 3.4905x; 1.6916x over previous
import jax
import jax.numpy as jnp
from jax import lax
from jax.experimental import pallas as pl
from jax.experimental.pallas import tpu as pltpu
from jax.experimental.pallas import tpu_sc as plsc

N = 10000
D = 128
H = 128
HH = H // 2
NACC = 10240
C = 128
NCHUNK = 84
NTILES = 32
EP = NTILES * NCHUNK * C
RPT = NACC // 16


def _tc_pre_body(x_ref, w_ref, asv_ref, adv_ref,
                 hlo_ref, hhi_ref, as_ref, ad_ref, m_ref):
    h = jnp.dot(x_ref[...], w_ref[...], preferred_element_type=jnp.float32)
    hlo_ref[...] = h[:, :HH]
    hhi_ref[...] = h[:, HH:]
    a_s = jnp.dot(h, asv_ref[...], preferred_element_type=jnp.float32)
    a_d = jnp.dot(h, adv_ref[...], preferred_element_type=jnp.float32)
    as_ref[...] = a_s
    ad_ref[...] = a_d
    m = jnp.maximum(jnp.max(a_s) + jnp.max(a_d), 0.0)
    m_ref[...] = jnp.reshape(m, (1, 1))


_tc_pre = pl.pallas_call(
    _tc_pre_body,
    out_shape=[
        jax.ShapeDtypeStruct((N, HH), jnp.float32),
        jax.ShapeDtypeStruct((N, HH), jnp.float32),
        jax.ShapeDtypeStruct((N, 1), jnp.float32),
        jax.ShapeDtypeStruct((N, 1), jnp.float32),
        jax.ShapeDtypeStruct((1, 1), jnp.float32),
    ],
)


def _tc_mid_body(alo_ref, ahi_ref, den_ref, b_ref, a_ref, w_ref,
                 asv_ref, adv_ref, hlo_ref, hhi_ref, as_ref, ad_ref, m_ref):
    dens = den_ref[0, :N, :] + den_ref[1, :N, :] + 1e-16
    vlo = (alo_ref[0, :N, :] + alo_ref[1, :N, :]) / dens + b_ref[:, :HH]
    vhi = (ahi_ref[0, :N, :] + ahi_ref[1, :N, :]) / dens + b_ref[:, HH:]
    a = a_ref[0, 0]
    f = jnp.concatenate(
        [jnp.where(vlo >= 0, vlo, a * vlo), jnp.where(vhi >= 0, vhi, a * vhi)],
        axis=1)
    h2 = jnp.dot(f, w_ref[...], preferred_element_type=jnp.float32)
    hlo_ref[...] = h2[:, :HH]
    hhi_ref[...] = h2[:, HH:]
    a_s = jnp.dot(h2, asv_ref[...], preferred_element_type=jnp.float32)
    a_d = jnp.dot(h2, adv_ref[...], preferred_element_type=jnp.float32)
    as_ref[...] = a_s
    ad_ref[...] = a_d
    m = jnp.maximum(jnp.max(a_s) + jnp.max(a_d), 0.0)
    m_ref[...] = jnp.reshape(m, (1, 1))


_tc_mid = pl.pallas_call(
    _tc_mid_body,
    compiler_params=pltpu.CompilerParams(vmem_limit_bytes=100 * 1024 * 1024),
    out_shape=[
        jax.ShapeDtypeStruct((N, HH), jnp.float32),
        jax.ShapeDtypeStruct((N, HH), jnp.float32),
        jax.ShapeDtypeStruct((N, 1), jnp.float32),
        jax.ShapeDtypeStruct((N, 1), jnp.float32),
        jax.ShapeDtypeStruct((1, 1), jnp.float32),
    ],
)


def _tc_post_body(alo_ref, ahi_ref, den_ref, b_ref, a_ref, out_ref):
    dens = den_ref[0, :N, :] + den_ref[1, :N, :] + 1e-16
    vlo = (alo_ref[0, :N, :] + alo_ref[1, :N, :]) / dens + b_ref[:, :HH]
    vhi = (ahi_ref[0, :N, :] + ahi_ref[1, :N, :]) / dens + b_ref[:, HH:]
    a = a_ref[0, 0]
    out_ref[...] = jnp.concatenate(
        [jnp.where(vlo >= 0, vlo, a * vlo), jnp.where(vhi >= 0, vhi, a * vhi)],
        axis=1)


_tc_post = pl.pallas_call(
    _tc_post_body,
    compiler_params=pltpu.CompilerParams(vmem_limit_bytes=100 * 1024 * 1024),
    out_shape=jax.ShapeDtypeStruct((N, H), jnp.float32),
)


def _sc_edge_body(src_hbm, dst_hbm, hlo_hbm, hhi_hbm, as_hbm, ad_hbm, m_hbm,
                  alo_out, ahi_out, den_out,
                  asbuf, adbuf, srcbuf, dstbuf, exall, rows_a, rows_b, rows_c,
                  zbuf, mbuf, acc_sh, den_sh,
                  sem_g0, sem_g1, sem_g2, sem_s0, sem_s1, sem_s2, sem_d):
    c = lax.axis_index("c")
    s = lax.axis_index("s")
    blk = c * 16 + s
    base = s * RPT

    zero16 = jnp.zeros((16,), jnp.float32)

    def _zero_rows_a():
        def _zrow(i, carry):
            for k in range(HH // 16):
                rows_a[i, pl.ds(k * 16, 16)] = zero16
            return carry
        lax.fori_loop(0, C, _zrow, 0)

    def _zero_acc_slice():
        for r in range(4):
            pltpu.sync_copy(rows_a, acc_sh.at[pl.ds(base + r * C, C)])
        pltpu.sync_copy(rows_a.at[pl.ds(0, RPT - 4 * C)],
                        acc_sh.at[pl.ds(base + 4 * C, RPT - 4 * C)])

    _zero_rows_a()
    _zero_acc_slice()
    for k in range(RPT // 16):
        zbuf[pl.ds(k * 16, 16)] = zero16
    pltpu.sync_copy(zbuf, den_sh.at[pl.ds(base, RPT)])

    pltpu.sync_copy(as_hbm, asbuf)
    pltpu.sync_copy(ad_hbm, adbuf)
    pltpu.sync_copy(src_hbm.at[blk], srcbuf)
    pltpu.sync_copy(dst_hbm.at[blk], dstbuf)
    pltpu.sync_copy(m_hbm, mbuf)
    plsc.subcore_barrier()
    mv = mbuf[...]

    rows = (rows_a, rows_b, rows_c)
    sg = (sem_g0, sem_g1, sem_g2)
    ss = (sem_s0, sem_s1, sem_s2)

    def _run_pass(h_hbm, compute_ex):
        def _phase(j, p):
            b2 = (p + 2) % 3

            @pl.when(j >= 1)
            def _():
                pltpu.make_async_copy(
                    rows[b2], acc_sh.at[dstbuf.at[j]], ss[b2]).wait()

            @pl.when(j + 2 < NCHUNK)
            def _():
                pltpu.async_copy(h_hbm.at[srcbuf.at[j + 2]], rows[b2], sg[b2])

            pltpu.make_async_copy(
                h_hbm.at[srcbuf.at[j]], rows[p], sg[p]).wait()

            if compute_ex:
                for g in range(C // 16):
                    sidx = srcbuf[j, pl.ds(g * 16, 16)]
                    didx = dstbuf[j, pl.ds(g * 16, 16)]
                    z = (plsc.load_gather(asbuf, [sidx])
                         + plsc.load_gather(adbuf, [didx]))
                    e = jnp.where(z >= 0, z, jnp.float32(0.2) * z)
                    exall[j, pl.ds(g * 16, 16)] = jnp.exp(e - mv)

                @pl.when(j >= 1)
                def _():
                    pltpu.make_async_copy(
                        exall.at[j], den_sh.at[dstbuf.at[j]], sem_d).wait()

                pltpu.async_copy(exall.at[j], den_sh.at[dstbuf.at[j]], sem_d,
                                 add=True)

            @plsc.parallel_loop(0, C // 16, unroll=2)
            def _scale(g):
                ex16 = exall[j, pl.ds(g * 16, 16)]
                for l in range(16):
                    ex = ex16[l]
                    i = g * 16 + l
                    for k in range(HH // 16):
                        rows[p][i, pl.ds(k * 16, 16)] = (
                            rows[p][i, pl.ds(k * 16, 16)] * ex)
            pltpu.async_copy(rows[p], acc_sh.at[dstbuf.at[j]], ss[p], add=True)

        pltpu.async_copy(h_hbm.at[srcbuf.at[0]], rows[0], sg[0])
        pltpu.async_copy(h_hbm.at[srcbuf.at[1]], rows[1], sg[1])

        def _outer(jj, carry):
            for p in range(3):
                _phase(jj * 3 + p, p)
            return carry

        lax.fori_loop(0, NCHUNK // 3, _outer, 0)
        pltpu.make_async_copy(
            rows[(NCHUNK - 1) % 3], acc_sh.at[dstbuf.at[0]],
            ss[(NCHUNK - 1) % 3]).wait()
        if compute_ex:
            pltpu.make_async_copy(
                exall.at[0], den_sh.at[dstbuf.at[0]], sem_d).wait()

    _run_pass(hlo_hbm, compute_ex=True)
    plsc.subcore_barrier()
    pltpu.sync_copy(acc_sh.at[pl.ds(base, RPT)],
                    alo_out.at[c].at[pl.ds(base, RPT)])
    pltpu.sync_copy(den_sh.at[pl.ds(base, RPT)],
                    den_out.at[c].at[pl.ds(base, RPT)])
    _zero_rows_a()
    _zero_acc_slice()
    plsc.subcore_barrier()
    _run_pass(hhi_hbm, compute_ex=False)
    plsc.subcore_barrier()
    pltpu.sync_copy(acc_sh.at[pl.ds(base, RPT)],
                    ahi_out.at[c].at[pl.ds(base, RPT)])


_sc_edge = pl.kernel(
    _sc_edge_body,
    out_type=[
        jax.ShapeDtypeStruct((2, NACC, HH), jnp.float32),
        jax.ShapeDtypeStruct((2, NACC, HH), jnp.float32),
        jax.ShapeDtypeStruct((2, NACC), jnp.float32),
    ],
    mesh=plsc.VectorSubcoreMesh(core_axis_name="c", subcore_axis_name="s",
                                num_cores=2, num_subcores=16),
    compiler_params=pltpu.CompilerParams(needs_layout_passes=False,
                                         use_tc_tiling_on_sc=False),
    scratch_types=[
        pltpu.VMEM((NACC,), jnp.float32),
        pltpu.VMEM((NACC,), jnp.float32),
        pltpu.VMEM((NCHUNK, C), jnp.int32),
        pltpu.VMEM((NCHUNK, C), jnp.int32),
        pltpu.VMEM((NCHUNK, C), jnp.float32),
        pltpu.VMEM((C, HH), jnp.float32),
        pltpu.VMEM((C, HH), jnp.float32),
        pltpu.VMEM((C, HH), jnp.float32),
        pltpu.VMEM((RPT,), jnp.float32),
        pltpu.VMEM((16,), jnp.float32),
        pltpu.VMEM_SHARED((NACC, HH), jnp.float32),
        pltpu.VMEM_SHARED((NACC,), jnp.float32),
        pltpu.SemaphoreType.DMA,
        pltpu.SemaphoreType.DMA,
        pltpu.SemaphoreType.DMA,
        pltpu.SemaphoreType.DMA,
        pltpu.SemaphoreType.DMA,
        pltpu.SemaphoreType.DMA,
        pltpu.SemaphoreType.DMA,
    ],
)


def _pad_nodes(a):
    return jnp.zeros((NACC,), jnp.float32).at[:N].set(a[:, 0])


def kernel(x, edge_index, W1, att_src1, att_dst1, b1, W2, att_src2, att_dst2,
           b2, prelu_a):
    loop = jnp.arange(N, dtype=jnp.int32)
    npad = EP - (edge_index.shape[1] + N)
    pad_src = jnp.arange(npad, dtype=jnp.int32) % N
    src = jnp.concatenate(
        [edge_index[0], loop, pad_src]
    ).reshape(NTILES, NCHUNK, C)
    pad_dst = N + jnp.arange(npad, dtype=jnp.int32) % (NACC - N)
    dst = jnp.concatenate(
        [edge_index[1], loop, pad_dst]
    ).reshape(NTILES, NCHUNK, C)

    a2 = prelu_a.reshape(1, 1)

    h1lo, h1hi, as1, ad1, m1 = _tc_pre(x, W1, att_src1.reshape(H, 1),
                                       att_dst1.reshape(H, 1))
    alo1, ahi1, den1 = _sc_edge(src, dst, h1lo, h1hi, _pad_nodes(as1),
                                _pad_nodes(ad1),
                                jnp.full((16,), m1[0, 0], jnp.float32))
    h2lo, h2hi, as2, ad2, m2 = _tc_mid(alo1, ahi1, den1.reshape(2, NACC, 1),
                                       b1.reshape(1, H), a2, W2,
                                       att_src2.reshape(H, 1),
                                       att_dst2.reshape(H, 1))
    alo2, ahi2, den2 = _sc_edge(src, dst, h2lo, h2hi, _pad_nodes(as2),
                                _pad_nodes(ad2),
                                jnp.full((16,), m2[0, 0], jnp.float32))
    return _tc_post(alo2, ahi2, den2.reshape(2, NACC, 1), b2.reshape(1, H), a2)


# speedup vs baseline: 3.5805x; 1.0258x over previous
import jax
import jax.numpy as jnp
from jax import lax
from jax.experimental import pallas as pl
from jax.experimental.pallas import tpu as pltpu
from jax.experimental.pallas import tpu_sc as plsc

N = 10000
D = 128
H = 128
HH = H // 2
NACC = 10240
C = 128
NCHUNK = 84
NTILES = 32
EP = NTILES * NCHUNK * C
RPT = NACC // 16


def _tc_pre_body(x_ref, w_ref, asv_ref, adv_ref,
                 hlo_ref, hhi_ref, as_ref, ad_ref, m_ref):
    h = jnp.dot(x_ref[...], w_ref[...], preferred_element_type=jnp.float32)
    hlo_ref[...] = h[:, :HH]
    hhi_ref[...] = h[:, HH:]
    a_s = jnp.dot(h, asv_ref[...], preferred_element_type=jnp.float32)
    a_d = jnp.dot(h, adv_ref[...], preferred_element_type=jnp.float32)
    as_ref[...] = a_s
    ad_ref[...] = a_d
    m = jnp.maximum(jnp.max(a_s) + jnp.max(a_d), 0.0)
    m_ref[...] = jnp.reshape(m, (1, 1))


_tc_pre = pl.pallas_call(
    _tc_pre_body,
    out_shape=[
        jax.ShapeDtypeStruct((N, HH), jnp.float32),
        jax.ShapeDtypeStruct((N, HH), jnp.float32),
        jax.ShapeDtypeStruct((N, 1), jnp.float32),
        jax.ShapeDtypeStruct((N, 1), jnp.float32),
        jax.ShapeDtypeStruct((1, 1), jnp.float32),
    ],
)


def _tc_mid_body(alo_ref, ahi_ref, den_ref, b_ref, a_ref, w_ref,
                 asv_ref, adv_ref, hlo_ref, hhi_ref, as_ref, ad_ref, m_ref):
    dens = den_ref[0, :N, :] + den_ref[1, :N, :] + 1e-16
    vlo = (alo_ref[0, :N, :] + alo_ref[1, :N, :]) / dens + b_ref[:, :HH]
    vhi = (ahi_ref[0, :N, :] + ahi_ref[1, :N, :]) / dens + b_ref[:, HH:]
    a = a_ref[0, 0]
    f = jnp.concatenate(
        [jnp.where(vlo >= 0, vlo, a * vlo), jnp.where(vhi >= 0, vhi, a * vhi)],
        axis=1)
    h2 = jnp.dot(f, w_ref[...], preferred_element_type=jnp.float32)
    hlo_ref[...] = h2[:, :HH]
    hhi_ref[...] = h2[:, HH:]
    a_s = jnp.dot(h2, asv_ref[...], preferred_element_type=jnp.float32)
    a_d = jnp.dot(h2, adv_ref[...], preferred_element_type=jnp.float32)
    as_ref[...] = a_s
    ad_ref[...] = a_d
    m = jnp.maximum(jnp.max(a_s) + jnp.max(a_d), 0.0)
    m_ref[...] = jnp.reshape(m, (1, 1))


_tc_mid = pl.pallas_call(
    _tc_mid_body,
    compiler_params=pltpu.CompilerParams(vmem_limit_bytes=100 * 1024 * 1024),
    out_shape=[
        jax.ShapeDtypeStruct((N, HH), jnp.float32),
        jax.ShapeDtypeStruct((N, HH), jnp.float32),
        jax.ShapeDtypeStruct((N, 1), jnp.float32),
        jax.ShapeDtypeStruct((N, 1), jnp.float32),
        jax.ShapeDtypeStruct((1, 1), jnp.float32),
    ],
)


def _tc_post_body(alo_ref, ahi_ref, den_ref, b_ref, a_ref, out_ref):
    dens = den_ref[0, :N, :] + den_ref[1, :N, :] + 1e-16
    vlo = (alo_ref[0, :N, :] + alo_ref[1, :N, :]) / dens + b_ref[:, :HH]
    vhi = (ahi_ref[0, :N, :] + ahi_ref[1, :N, :]) / dens + b_ref[:, HH:]
    a = a_ref[0, 0]
    out_ref[...] = jnp.concatenate(
        [jnp.where(vlo >= 0, vlo, a * vlo), jnp.where(vhi >= 0, vhi, a * vhi)],
        axis=1)


_tc_post = pl.pallas_call(
    _tc_post_body,
    compiler_params=pltpu.CompilerParams(vmem_limit_bytes=100 * 1024 * 1024),
    out_shape=jax.ShapeDtypeStruct((N, H), jnp.float32),
)


def _sc_edge_body(src_hbm, dst_hbm, hlo_hbm, hhi_hbm, as_hbm, ad_hbm, m_hbm,
                  alo_out, ahi_out, den_out,
                  asbuf, adbuf, srcbuf, dstbuf, exall, rows_a, rows_b, rows_c,
                  zbuf, mbuf, acc_sh, den_sh,
                  sem_g0, sem_g1, sem_g2, sem_s0, sem_s1, sem_s2, sem_d):
    c = lax.axis_index("c")
    s = lax.axis_index("s")
    blk = c * 16 + s
    base = s * RPT

    zero16 = jnp.zeros((16,), jnp.float32)

    def _zero_rows_a():
        def _zrow(i, carry):
            for k in range(HH // 16):
                rows_a[i, pl.ds(k * 16, 16)] = zero16
            return carry
        lax.fori_loop(0, C, _zrow, 0)

    def _zero_acc_slice():
        for r in range(4):
            pltpu.sync_copy(rows_a, acc_sh.at[pl.ds(base + r * C, C)])
        pltpu.sync_copy(rows_a.at[pl.ds(0, RPT - 4 * C)],
                        acc_sh.at[pl.ds(base + 4 * C, RPT - 4 * C)])

    _zero_rows_a()
    _zero_acc_slice()
    for k in range(RPT // 16):
        zbuf[pl.ds(k * 16, 16)] = zero16
    pltpu.sync_copy(zbuf, den_sh.at[pl.ds(base, RPT)])

    pltpu.sync_copy(as_hbm, asbuf)
    pltpu.sync_copy(ad_hbm, adbuf)
    pltpu.sync_copy(src_hbm.at[blk], srcbuf)
    pltpu.sync_copy(dst_hbm.at[blk], dstbuf)
    pltpu.sync_copy(m_hbm, mbuf)
    plsc.subcore_barrier()
    mv = mbuf[...]

    rows = (rows_a, rows_b, rows_c)
    sg = (sem_g0, sem_g1, sem_g2)
    ss = (sem_s0, sem_s1, sem_s2)

    def _run_pass(h_hbm, compute_ex):
        def _phase(j, p):
            b2 = (p + 2) % 3

            @pl.when(j >= 1)
            def _():
                pltpu.make_async_copy(
                    rows[b2], acc_sh.at[dstbuf.at[j]], ss[b2]).wait()

            @pl.when(j + 2 < NCHUNK)
            def _():
                pltpu.async_copy(h_hbm.at[srcbuf.at[j + 2]], rows[b2], sg[b2])

            pltpu.make_async_copy(
                h_hbm.at[srcbuf.at[j]], rows[p], sg[p]).wait()

            if compute_ex:
                @plsc.parallel_loop(0, C // 16, unroll=2)
                def _exloop(g):
                    sidx = srcbuf[j, pl.ds(g * 16, 16)]
                    didx = dstbuf[j, pl.ds(g * 16, 16)]
                    z = (plsc.load_gather(asbuf, [sidx])
                         + plsc.load_gather(adbuf, [didx]))
                    e = jnp.where(z >= 0, z, jnp.float32(0.2) * z)
                    exall[j, pl.ds(g * 16, 16)] = jnp.exp(e - mv)

                @pl.when(j >= 1)
                def _():
                    pltpu.make_async_copy(
                        exall.at[j], den_sh.at[dstbuf.at[j]], sem_d).wait()

                pltpu.async_copy(exall.at[j], den_sh.at[dstbuf.at[j]], sem_d,
                                 add=True)

            @plsc.parallel_loop(0, C // 16, unroll=4)
            def _scale(g):
                ex16 = exall[j, pl.ds(g * 16, 16)]
                for l in range(16):
                    ex = ex16[l]
                    i = g * 16 + l
                    for k in range(HH // 16):
                        rows[p][i, pl.ds(k * 16, 16)] = (
                            rows[p][i, pl.ds(k * 16, 16)] * ex)
            pltpu.async_copy(rows[p], acc_sh.at[dstbuf.at[j]], ss[p], add=True)

        pltpu.async_copy(h_hbm.at[srcbuf.at[0]], rows[0], sg[0])
        pltpu.async_copy(h_hbm.at[srcbuf.at[1]], rows[1], sg[1])

        def _outer(jj, carry):
            for p in range(3):
                _phase(jj * 3 + p, p)
            return carry

        lax.fori_loop(0, NCHUNK // 3, _outer, 0)
        pltpu.make_async_copy(
            rows[(NCHUNK - 1) % 3], acc_sh.at[dstbuf.at[0]],
            ss[(NCHUNK - 1) % 3]).wait()
        if compute_ex:
            pltpu.make_async_copy(
                exall.at[0], den_sh.at[dstbuf.at[0]], sem_d).wait()

    _run_pass(hlo_hbm, compute_ex=True)
    plsc.subcore_barrier()
    pltpu.sync_copy(acc_sh.at[pl.ds(base, RPT)],
                    alo_out.at[c].at[pl.ds(base, RPT)])
    pltpu.sync_copy(den_sh.at[pl.ds(base, RPT)],
                    den_out.at[c].at[pl.ds(base, RPT)])
    _zero_rows_a()
    _zero_acc_slice()
    plsc.subcore_barrier()
    _run_pass(hhi_hbm, compute_ex=False)
    plsc.subcore_barrier()
    pltpu.sync_copy(acc_sh.at[pl.ds(base, RPT)],
                    ahi_out.at[c].at[pl.ds(base, RPT)])


_sc_edge = pl.kernel(
    _sc_edge_body,
    out_type=[
        jax.ShapeDtypeStruct((2, NACC, HH), jnp.float32),
        jax.ShapeDtypeStruct((2, NACC, HH), jnp.float32),
        jax.ShapeDtypeStruct((2, NACC), jnp.float32),
    ],
    mesh=plsc.VectorSubcoreMesh(core_axis_name="c", subcore_axis_name="s",
                                num_cores=2, num_subcores=16),
    compiler_params=pltpu.CompilerParams(needs_layout_passes=False,
                                         use_tc_tiling_on_sc=False),
    scratch_types=[
        pltpu.VMEM((NACC,), jnp.float32),
        pltpu.VMEM((NACC,), jnp.float32),
        pltpu.VMEM((NCHUNK, C), jnp.int32),
        pltpu.VMEM((NCHUNK, C), jnp.int32),
        pltpu.VMEM((NCHUNK, C), jnp.float32),
        pltpu.VMEM((C, HH), jnp.float32),
        pltpu.VMEM((C, HH), jnp.float32),
        pltpu.VMEM((C, HH), jnp.float32),
        pltpu.VMEM((RPT,), jnp.float32),
        pltpu.VMEM((16,), jnp.float32),
        pltpu.VMEM_SHARED((NACC, HH), jnp.float32),
        pltpu.VMEM_SHARED((NACC,), jnp.float32),
        pltpu.SemaphoreType.DMA,
        pltpu.SemaphoreType.DMA,
        pltpu.SemaphoreType.DMA,
        pltpu.SemaphoreType.DMA,
        pltpu.SemaphoreType.DMA,
        pltpu.SemaphoreType.DMA,
        pltpu.SemaphoreType.DMA,
    ],
)


def _pad_nodes(a):
    return jnp.zeros((NACC,), jnp.float32).at[:N].set(a[:, 0])


def kernel(x, edge_index, W1, att_src1, att_dst1, b1, W2, att_src2, att_dst2,
           b2, prelu_a):
    loop = jnp.arange(N, dtype=jnp.int32)
    npad = EP - (edge_index.shape[1] + N)
    pad_src = jnp.arange(npad, dtype=jnp.int32) % N
    src = jnp.concatenate(
        [edge_index[0], loop, pad_src]
    ).reshape(NTILES, NCHUNK, C)
    pad_dst = N + jnp.arange(npad, dtype=jnp.int32) % (NACC - N)
    dst = jnp.concatenate(
        [edge_index[1], loop, pad_dst]
    ).reshape(NTILES, NCHUNK, C)

    a2 = prelu_a.reshape(1, 1)

    h1lo, h1hi, as1, ad1, m1 = _tc_pre(x, W1, att_src1.reshape(H, 1),
                                       att_dst1.reshape(H, 1))
    alo1, ahi1, den1 = _sc_edge(src, dst, h1lo, h1hi, _pad_nodes(as1),
                                _pad_nodes(ad1),
                                jnp.full((16,), m1[0, 0], jnp.float32))
    h2lo, h2hi, as2, ad2, m2 = _tc_mid(alo1, ahi1, den1.reshape(2, NACC, 1),
                                       b1.reshape(1, H), a2, W2,
                                       att_src2.reshape(H, 1),
                                       att_dst2.reshape(H, 1))
    alo2, ahi2, den2 = _sc_edge(src, dst, h2lo, h2hi, _pad_nodes(as2),
                                _pad_nodes(ad2),
                                jnp.full((16,), m2[0, 0], jnp.float32))
    return _tc_post(alo2, ahi2, den2.reshape(2, NACC, 1), b2.reshape(1, H), a2)
